# baseline (speedup 1.0000x reference)
import functools

import jax
import jax.numpy as jnp
from jax import lax
from jax.experimental import pallas as pl
from jax.experimental.pallas import tpu as pltpu
from jax.experimental.pallas import tpu_sc as plsc

_N_NODES = 10000
_D = 256
_DP = _D // 2
_N_EDGES = 160000
_CHUNK = 64
_N_CHUNKS = _N_EDGES // _CHUNK
_NW = 32
_BASE = _N_CHUNKS // _NW
_EXTRA = _N_CHUNKS % _NW
_IDXCAP = (_BASE + 2) * _CHUNK


def _sigpack_body(x_ref, o_ref):
    s = jax.nn.sigmoid(x_ref[...])
    lo = jax.lax.bitcast_convert_type(
        s[:, :_DP].astype(jnp.bfloat16), jnp.uint16).astype(jnp.uint32)
    hi = jax.lax.bitcast_convert_type(
        s[:, _DP:].astype(jnp.bfloat16), jnp.uint16).astype(jnp.uint32)
    o_ref[...] = jax.lax.bitcast_convert_type(lo | (hi << 16), jnp.int32)


def _sigmoid_pack_table(x):
    n, d = x.shape
    blk = 2000
    return pl.pallas_call(
        _sigpack_body,
        grid=(n // blk,),
        in_specs=[pl.BlockSpec((blk, d), lambda i: (i, 0))],
        out_specs=pl.BlockSpec((blk, d // 2), lambda i: (i, 0)),
        out_shape=jax.ShapeDtypeStruct((n, d // 2), jnp.int32),
    )(x)


@functools.partial(
    pl.kernel,
    mesh=plsc.VectorSubcoreMesh(core_axis_name="c", subcore_axis_name="s"),
    out_type=jax.ShapeDtypeStruct((_N_EDGES, 2 * _D), jnp.float32),
    scratch_types=[
        pltpu.VMEM((_IDXCAP,), jnp.int32),
        pltpu.VMEM((_IDXCAP,), jnp.int32),
        pltpu.VMEM((_CHUNK, _DP), jnp.int32),
        pltpu.VMEM((_CHUNK, _DP), jnp.int32),
        pltpu.VMEM((_CHUNK, _DP), jnp.int32),
        pltpu.VMEM((_CHUNK, _DP), jnp.int32),
        pltpu.VMEM((_CHUNK, 2 * _D), jnp.float32),
        pltpu.VMEM((_CHUNK, 2 * _D), jnp.float32),
        pltpu.SemaphoreType.DMA,
        pltpu.SemaphoreType.DMA,
        pltpu.SemaphoreType.DMA,
        pltpu.SemaphoreType.DMA,
    ],
)
def _gather_rows(s_hbm, edge_hbm, out_hbm, idxs_v, idxd_v,
                 gs0, gs1, gd0, gd1, f0, f1, g0, g1, o0, o1):
    gsrc = (gs0, gs1)
    gdst = (gd0, gd1)
    fbuf = (f0, f1)
    gsem = (g0, g1)
    osem = (o0, o1)

    wid = lax.axis_index("s") * 2 + lax.axis_index("c")
    start = wid * _BASE + jnp.minimum(wid, _EXTRA)
    has_extra = wid < _EXTRA
    n = _BASE + has_extra.astype(jnp.int32)

    start_al = jnp.minimum((start // 2) * 2, (_N_EDGES - _IDXCAP) // _CHUNK)
    off = start - start_al
    pltpu.async_copy(
        edge_hbm.at[0, pl.ds(start_al * _CHUNK, _IDXCAP)], idxs_v, g0)
    pltpu.async_copy(
        edge_hbm.at[1, pl.ds(start_al * _CHUNK, _IDXCAP)], idxd_v, g1)
    pltpu.make_async_copy(
        edge_hbm.at[0, pl.ds(start_al * _CHUNK, _IDXCAP)], idxs_v, g0).wait()
    pltpu.make_async_copy(
        edge_hbm.at[1, pl.ds(start_al * _CHUNK, _IDXCAP)], idxd_v, g1).wait()

    def start_gather(j, b):
        sl = pl.ds((off + j) * _CHUNK, _CHUNK)
        pltpu.async_copy(s_hbm.at[idxs_v.at[sl]], gsrc[b], gsem[b])
        pltpu.async_copy(s_hbm.at[idxd_v.at[sl]], gdst[b], gsem[b])

    def wait_gather(j, b):
        sl = pl.ds((off + j) * _CHUNK, _CHUNK)
        pltpu.make_async_copy(s_hbm.at[idxs_v.at[sl]], gsrc[b], gsem[b]).wait()
        pltpu.make_async_copy(s_hbm.at[idxd_v.at[sl]], gdst[b], gsem[b]).wait()

    def expand(b):
        hmask = jnp.int32(-65536)

        def row(r, acc):
            for half, gbuf in ((0, gsrc[b]), (1, gdst[b])):
                base = half * _D
                for p in range(_DP // 16):
                    w = gbuf[r, pl.ds(p * 16, 16)]
                    lo = jax.lax.bitcast_convert_type(
                        jax.lax.shift_left(w, 16), jnp.float32)
                    hi = jax.lax.bitcast_convert_type(w & hmask, jnp.float32)
                    fbuf[b][r, pl.ds(base + p * 16, 16)] = lo
                    fbuf[b][r, pl.ds(base + _DP + p * 16, 16)] = hi
            return acc

        lax.fori_loop(0, _CHUNK, row, 0)

    def start_scatter(j, b):
        pltpu.async_copy(
            fbuf[b], out_hbm.at[pl.ds((start + j) * _CHUNK, _CHUNK)], osem[b])

    def wait_scatter(b):
        pltpu.make_async_copy(
            fbuf[b], out_hbm.at[pl.ds(0, _CHUNK)], osem[b]).wait()

    for b in range(2):
        start_gather(b, b)

    def pair(t, carry):
        for b in range(2):
            k = 2 * t + b
            wait_gather(k, b)
            expand(b)
            start_scatter(k, b)

            @pl.when(k + 2 < n)
            def _():
                wait_scatter(b)
                start_gather(k + 2, b)

        return carry

    lax.fori_loop(0, _BASE // 2, pair, 0)

    @pl.when(has_extra)
    def _():
        wait_gather(_BASE, _BASE % 2)
        expand(_BASE % 2)
        start_scatter(_BASE, _BASE % 2)

    for b in range(2):
        wait_scatter(b)


def kernel(x, edge_index):
    sp = _sigmoid_pack_table(x)
    return _gather_rows(sp, edge_index.astype(jnp.int32))


# speedup vs baseline: 1.2462x; 1.2462x over previous
import functools

import jax
import jax.numpy as jnp
from jax import lax
from jax.experimental import pallas as pl
from jax.experimental.pallas import tpu as pltpu
from jax.experimental.pallas import tpu_sc as plsc

_N_NODES = 10000
_D = 256
_DP = _D // 2
_N_EDGES = 160000
_CHUNK = 64
_N_CHUNKS = _N_EDGES // _CHUNK
_NW = 32
_BASE = _N_CHUNKS // _NW
_EXTRA = _N_CHUNKS % _NW
_IDXCAP = (_BASE + 2) * _CHUNK


def _sigpack_body(x_ref, o_ref):
    s = jax.nn.sigmoid(x_ref[...])
    lo = jax.lax.bitcast_convert_type(
        s[:, :_DP].astype(jnp.bfloat16), jnp.uint16).astype(jnp.uint32)
    hi = jax.lax.bitcast_convert_type(
        s[:, _DP:].astype(jnp.bfloat16), jnp.uint16).astype(jnp.uint32)
    o_ref[...] = jax.lax.bitcast_convert_type(lo | (hi << 16), jnp.int32)


def _sigmoid_pack_table(x):
    n, d = x.shape
    blk = 2000
    return pl.pallas_call(
        _sigpack_body,
        grid=(n // blk,),
        in_specs=[pl.BlockSpec((blk, d), lambda i: (i, 0))],
        out_specs=pl.BlockSpec((blk, d // 2), lambda i: (i, 0)),
        out_shape=jax.ShapeDtypeStruct((n, d // 2), jnp.int32),
    )(x)


@functools.partial(
    pl.kernel,
    mesh=plsc.VectorSubcoreMesh(core_axis_name="c", subcore_axis_name="s"),
    out_type=jax.ShapeDtypeStruct((_N_EDGES, 2 * _D), jnp.float32),
    scratch_types=[
        pltpu.VMEM((_IDXCAP,), jnp.int32),
        pltpu.VMEM((_IDXCAP,), jnp.int32),
        pltpu.VMEM((_CHUNK, _DP), jnp.int32),
        pltpu.VMEM((_CHUNK, _DP), jnp.int32),
        pltpu.VMEM((_CHUNK, _DP), jnp.int32),
        pltpu.VMEM((_CHUNK, _DP), jnp.int32),
        pltpu.VMEM((_CHUNK, 2 * _D), jnp.float32),
        pltpu.VMEM((_CHUNK, 2 * _D), jnp.float32),
        pltpu.SemaphoreType.DMA,
        pltpu.SemaphoreType.DMA,
        pltpu.SemaphoreType.DMA,
        pltpu.SemaphoreType.DMA,
    ],
)
def _gather_rows(s_hbm, edge_hbm, out_hbm, idxs_v, idxd_v,
                 gs0, gs1, gd0, gd1, f0, f1, g0, g1, o0, o1):
    gsrc = (gs0, gs1)
    gdst = (gd0, gd1)
    fbuf = (f0, f1)
    gsem = (g0, g1)
    osem = (o0, o1)

    wid = lax.axis_index("s") * 2 + lax.axis_index("c")
    start = wid * _BASE + jnp.minimum(wid, _EXTRA)
    has_extra = wid < _EXTRA
    n = _BASE + has_extra.astype(jnp.int32)

    start_al = jnp.minimum((start // 2) * 2, (_N_EDGES - _IDXCAP) // _CHUNK)
    off = start - start_al
    pltpu.async_copy(
        edge_hbm.at[0, pl.ds(start_al * _CHUNK, _IDXCAP)], idxs_v, g0)
    pltpu.async_copy(
        edge_hbm.at[1, pl.ds(start_al * _CHUNK, _IDXCAP)], idxd_v, g1)
    pltpu.make_async_copy(
        edge_hbm.at[0, pl.ds(start_al * _CHUNK, _IDXCAP)], idxs_v, g0).wait()
    pltpu.make_async_copy(
        edge_hbm.at[1, pl.ds(start_al * _CHUNK, _IDXCAP)], idxd_v, g1).wait()

    def start_gather(j, b):
        sl = pl.ds((off + j) * _CHUNK, _CHUNK)
        pltpu.async_copy(s_hbm.at[idxs_v.at[sl]], gsrc[b], gsem[b])
        pltpu.async_copy(s_hbm.at[idxd_v.at[sl]], gdst[b], gsem[b])

    def wait_gather(j, b):
        sl = pl.ds((off + j) * _CHUNK, _CHUNK)
        pltpu.make_async_copy(s_hbm.at[idxs_v.at[sl]], gsrc[b], gsem[b]).wait()
        pltpu.make_async_copy(s_hbm.at[idxd_v.at[sl]], gdst[b], gsem[b]).wait()

    def expand(b):
        hmask = jnp.int32(-65536)
        unroll = 4

        def rows(t, acc):
            for u in range(unroll):
                r = unroll * t + u
                for half, gbuf in ((0, gsrc[b]), (1, gdst[b])):
                    base = half * _D
                    for p in range(_DP // 16):
                        w = gbuf[r, pl.ds(p * 16, 16)]
                        lo = jax.lax.bitcast_convert_type(
                            jax.lax.shift_left(w, 16), jnp.float32)
                        hi = jax.lax.bitcast_convert_type(
                            w & hmask, jnp.float32)
                        fbuf[b][r, pl.ds(base + p * 16, 16)] = lo
                        fbuf[b][r, pl.ds(base + _DP + p * 16, 16)] = hi
            return acc

        lax.fori_loop(0, _CHUNK // unroll, rows, 0)

    def start_scatter(j, b):
        pltpu.async_copy(
            fbuf[b], out_hbm.at[pl.ds((start + j) * _CHUNK, _CHUNK)], osem[b])

    def wait_scatter(b):
        pltpu.make_async_copy(
            fbuf[b], out_hbm.at[pl.ds(0, _CHUNK)], osem[b]).wait()

    for b in range(2):
        start_gather(b, b)

    def pair(t, carry):
        for b in range(2):
            k = 2 * t + b
            wait_gather(k, b)

            @pl.when(k >= 2)
            def _():
                wait_scatter(b)

            expand(b)
            start_scatter(k, b)

            @pl.when(k + 2 < n)
            def _():
                start_gather(k + 2, b)

        return carry

    lax.fori_loop(0, _BASE // 2, pair, 0)

    @pl.when(has_extra)
    def _():
        b = _BASE % 2
        wait_gather(_BASE, b)
        wait_scatter(b)
        expand(b)
        start_scatter(_BASE, b)

    for b in range(2):
        wait_scatter(b)


def kernel(x, edge_index):
    sp = _sigmoid_pack_table(x)
    return _gather_rows(sp, edge_index.astype(jnp.int32))


# speedup vs baseline: 2.3769x; 1.9073x over previous
import functools

import jax
import jax.numpy as jnp
from jax import lax
from jax.experimental import pallas as pl
from jax.experimental.pallas import tpu as pltpu
from jax.experimental.pallas import tpu_sc as plsc

_N_NODES = 10000
_D = 256
_DP = _D // 2
_N_EDGES = 160000
_CHUNK = 64
_N_CHUNKS = _N_EDGES // _CHUNK
_NW = 32
_BASE = _N_CHUNKS // _NW
_EXTRA = _N_CHUNKS % _NW
_IDXCAP = (_BASE + 2) * _CHUNK


def _sigpack_body(x_ref, o_ref):
    s = jax.nn.sigmoid(x_ref[...])
    lo = jax.lax.bitcast_convert_type(
        s[:, :_DP].astype(jnp.bfloat16), jnp.uint16).astype(jnp.uint32)
    hi = jax.lax.bitcast_convert_type(
        s[:, _DP:].astype(jnp.bfloat16), jnp.uint16).astype(jnp.uint32)
    o_ref[...] = jax.lax.bitcast_convert_type(lo | (hi << 16), jnp.int32)


def _sigmoid_pack_table(x):
    n, d = x.shape
    blk = 2000
    return pl.pallas_call(
        _sigpack_body,
        grid=(n // blk,),
        in_specs=[pl.BlockSpec((blk, d), lambda i: (i, 0))],
        out_specs=pl.BlockSpec((blk, d // 2), lambda i: (i, 0)),
        out_shape=jax.ShapeDtypeStruct((n, d // 2), jnp.int32),
    )(x)


@functools.partial(
    pl.kernel,
    mesh=plsc.VectorSubcoreMesh(core_axis_name="c", subcore_axis_name="s"),
    out_type=jax.ShapeDtypeStruct((_N_EDGES, 2 * _D), jnp.float32),
    scratch_types=[
        pltpu.VMEM((_IDXCAP,), jnp.int32),
        pltpu.VMEM((_IDXCAP,), jnp.int32),
        pltpu.VMEM((_CHUNK, _DP), jnp.int32),
        pltpu.VMEM((_CHUNK, _DP), jnp.int32),
        pltpu.VMEM((_CHUNK, _DP), jnp.int32),
        pltpu.VMEM((_CHUNK, _DP), jnp.int32),
        pltpu.VMEM((_CHUNK, 2 * _D), jnp.float32),
        pltpu.VMEM((_CHUNK, 2 * _D), jnp.float32),
        pltpu.SemaphoreType.DMA,
        pltpu.SemaphoreType.DMA,
        pltpu.SemaphoreType.DMA,
        pltpu.SemaphoreType.DMA,
    ],
)
def _gather_rows(s_hbm, edge_hbm, out_hbm, idxs_v, idxd_v,
                 gs0, gs1, gd0, gd1, f0, f1, g0, g1, o0, o1):
    gsrc = (gs0, gs1)
    gdst = (gd0, gd1)
    fbuf = (f0, f1)
    gsem = (g0, g1)
    osem = (o0, o1)

    wid = lax.axis_index("s") * 2 + lax.axis_index("c")
    start = wid * _BASE + jnp.minimum(wid, _EXTRA)
    has_extra = wid < _EXTRA
    n = _BASE + has_extra.astype(jnp.int32)

    start_al = jnp.minimum((start // 2) * 2, (_N_EDGES - _IDXCAP) // _CHUNK)
    off = start - start_al
    pltpu.async_copy(
        edge_hbm.at[0, pl.ds(start_al * _CHUNK, _IDXCAP)], idxs_v, g0)
    pltpu.async_copy(
        edge_hbm.at[1, pl.ds(start_al * _CHUNK, _IDXCAP)], idxd_v, g1)
    pltpu.make_async_copy(
        edge_hbm.at[0, pl.ds(start_al * _CHUNK, _IDXCAP)], idxs_v, g0).wait()
    pltpu.make_async_copy(
        edge_hbm.at[1, pl.ds(start_al * _CHUNK, _IDXCAP)], idxd_v, g1).wait()

    def start_gather(j, b):
        sl = pl.ds((off + j) * _CHUNK, _CHUNK)
        pltpu.async_copy(s_hbm.at[idxs_v.at[sl]], gsrc[b], gsem[b])
        pltpu.async_copy(s_hbm.at[idxd_v.at[sl]], gdst[b], gsem[b])

    def wait_gather(j, b):
        sl = pl.ds((off + j) * _CHUNK, _CHUNK)
        pltpu.make_async_copy(s_hbm.at[idxs_v.at[sl]], gsrc[b], gsem[b]).wait()
        pltpu.make_async_copy(s_hbm.at[idxd_v.at[sl]], gdst[b], gsem[b]).wait()

    def expand(b):
        hmask = jnp.int32(-65536)

        @plsc.parallel_loop(0, _CHUNK, 1, unroll=4)
        def _row(r):
            for half, gbuf in ((0, gsrc[b]), (1, gdst[b])):
                base = half * _D
                for p in range(_DP // 16):
                    w = gbuf[r, pl.ds(p * 16, 16)]
                    lo = jax.lax.bitcast_convert_type(
                        jax.lax.shift_left(w, 16), jnp.float32)
                    hi = jax.lax.bitcast_convert_type(w & hmask, jnp.float32)
                    fbuf[b][r, pl.ds(base + p * 16, 16)] = lo
                    fbuf[b][r, pl.ds(base + _DP + p * 16, 16)] = hi

    def start_scatter(j, b):
        pltpu.async_copy(
            fbuf[b], out_hbm.at[pl.ds((start + j) * _CHUNK, _CHUNK)], osem[b])

    def wait_scatter(b):
        pltpu.make_async_copy(
            fbuf[b], out_hbm.at[pl.ds(0, _CHUNK)], osem[b]).wait()

    for b in range(2):
        start_gather(b, b)

    def pair(t, carry):
        for b in range(2):
            k = 2 * t + b
            wait_gather(k, b)

            @pl.when(k >= 2)
            def _():
                wait_scatter(b)

            expand(b)
            start_scatter(k, b)

            @pl.when(k + 2 < n)
            def _():
                start_gather(k + 2, b)

        return carry

    lax.fori_loop(0, _BASE // 2, pair, 0)

    @pl.when(has_extra)
    def _():
        b = _BASE % 2
        wait_gather(_BASE, b)
        wait_scatter(b)
        expand(b)
        start_scatter(_BASE, b)

    for b in range(2):
        wait_scatter(b)


def kernel(x, edge_index):
    sp = _sigmoid_pack_table(x)
    return _gather_rows(sp, edge_index.astype(jnp.int32))
